# Initial kernel scaffold; baseline (speedup 1.0000x reference)
import jax
import jax.numpy as jnp
from jax.experimental import pallas as pl


def kernel(input_ids, word_table):
    raise NotImplementedError("write your pallas kernel here")


# speedup vs baseline: 3.6080x; 3.6080x over previous
import functools

import jax
import jax.numpy as jnp
from jax import lax
from jax.experimental import pallas as pl
from jax.experimental.pallas import tpu as pltpu
from jax.experimental.pallas import tpu_sc as plsc

_NC = 2
_NS = 16
_LANES = 16
_NW = _NC * _NS


def _row_sums_sc(ids_flat, table, b, s, d):
  bpw = b // _NW
  chunks = []
  off = 0
  while off < s:
    sz = min(128, s - off)
    chunks.append((off, sz))
    off += sz
  nacc = d // _LANES

  mesh = plsc.VectorSubcoreMesh(
      core_axis_name="c", subcore_axis_name="s", num_cores=_NC,
      num_subcores=_NS)

  @functools.partial(
      pl.kernel,
      out_type=jax.ShapeDtypeStruct((b, d), jnp.float32),
      mesh=mesh,
      scratch_types=(
          [pltpu.VMEM((sz,), jnp.int32) for _, sz in chunks]
          + [pltpu.VMEM((s, d), jnp.float32),
             pltpu.VMEM((1, d), jnp.float32),
             pltpu.SemaphoreType.DMA]),
  )
  def sums_kernel(ids_hbm, table_hbm, out_hbm, *scratch):
    idx_refs = scratch[:len(chunks)]
    buf, acc, sem = scratch[len(chunks):]
    wid = lax.axis_index("s") * _NC + lax.axis_index("c")
    row0 = wid * bpw

    def row_body(r, carry):
      row = row0 + r
      base = pl.multiple_of(row * s, 8)
      for (coff, csz), idx in zip(chunks, idx_refs):
        pltpu.sync_copy(ids_hbm.at[pl.ds(base + coff, csz)], idx)
      cps = [
          pltpu.async_copy(table_hbm.at[idx], buf.at[pl.ds(coff, csz)], sem)
          for (coff, csz), idx in zip(chunks, idx_refs)
      ]
      for cp in cps:
        cp.wait()

      def acc_body(j, accs):
        return tuple(accs[h] + buf[j, pl.ds(h * _LANES, _LANES)]
                     for h in range(nacc))

      accs = lax.fori_loop(
          0, s, acc_body,
          tuple(jnp.zeros((_LANES,), jnp.float32) for _ in range(nacc)))
      for h in range(nacc):
        acc[0, pl.ds(h * _LANES, _LANES)] = accs[h]
      pltpu.sync_copy(acc, out_hbm.at[pl.ds(row, 1)])
      return carry

    lax.fori_loop(0, bpw, row_body, 0)

  return sums_kernel(ids_flat, table)


def _post_tc(sums, input_ids, t0, b, s, h):
  bb = 256
  grid = (b // bb,)
  inv_s = 1.0 / s

  def post_kernel(sums_ref, ids_ref, t0_ref, outr_ref, outi_ref, prob_ref,
                  phase_ref, ampr_ref, ampi_ref):
    ids = ids_ref[...]
    nz = jnp.sum((ids == 0).astype(jnp.float32), axis=1, keepdims=True)
    cnt = s - nz
    tot = sums_ref[...]
    tr = tot[:, :h]
    ti = tot[:, h:]
    t0v = t0_ref[...]
    t0r = t0v[:, :h]
    t0i = t0v[:, h:]
    good = (cnt > 0.0).astype(jnp.float32)
    scale = good / (cnt + 1e-9)
    pr = (tr - nz * t0r) * scale
    pi = (ti - nz * t0i) * scale
    s2 = jnp.sum(pr * pr + pi * pi, axis=1, keepdims=True)
    norm = jnp.sqrt(s2) + 1e-9
    outr = pr / norm
    outi = pi / norm
    outr_ref[...] = outr
    outi_ref[...] = outi
    prob_ref[...] = outr * outr + outi * outi
    phase_ref[...] = jnp.arctan2(outi, outr)
    ampr_ref[...] = tr * inv_s
    ampi_ref[...] = ti * inv_s

  d = 2 * h
  out_spec = pl.BlockSpec((bb, h), lambda i: (i, 0))
  return pl.pallas_call(
      post_kernel,
      grid=grid,
      in_specs=[
          pl.BlockSpec((bb, d), lambda i: (i, 0)),
          pl.BlockSpec((bb, s), lambda i: (i, 0)),
          pl.BlockSpec((1, d), lambda i: (0, 0)),
      ],
      out_specs=[out_spec] * 6,
      out_shape=[jax.ShapeDtypeStruct((b, h), jnp.float32)] * 6,
  )(sums, input_ids, t0)


def kernel(input_ids, word_table):
  b, s = input_ids.shape
  v, d = word_table.shape
  h = d // 2
  ids_flat = input_ids.reshape(-1).astype(jnp.int32)
  sums = _row_sums_sc(ids_flat, word_table, b, s, d)
  outr, outi, prob, phase, ampr, ampi = _post_tc(
      sums, input_ids.astype(jnp.int32), word_table[0:1], b, s, h)
  state = jax.lax.complex(outr, outi)
  amplitudes = jnp.stack([ampr, ampi], axis=-1)
  return (state, amplitudes, prob, phase)


# speedup vs baseline: 7.1034x; 1.9688x over previous
import functools

import jax
import jax.numpy as jnp
from jax import lax
from jax.experimental import pallas as pl
from jax.experimental.pallas import tpu as pltpu
from jax.experimental.pallas import tpu_sc as plsc

_NC = 2
_NS = 16
_LANES = 16
_NW = _NC * _NS


def _row_sums_sc(ids_flat, table, b, s, d):
  bpw = b // _NW
  half = bpw // 2
  chunks = []
  off = 0
  while off < s:
    sz = min(128, s - off)
    chunks.append((off, sz))
    off += sz
  nacc = d // _LANES

  mesh = plsc.VectorSubcoreMesh(
      core_axis_name="c", subcore_axis_name="s", num_cores=_NC,
      num_subcores=_NS)

  @functools.partial(
      pl.kernel,
      out_type=jax.ShapeDtypeStruct((b, d), jnp.float32),
      mesh=mesh,
      scratch_types=[
          pltpu.VMEM((bpw * s,), jnp.int32),
          pltpu.VMEM((s, d), jnp.float32),
          pltpu.VMEM((s, d), jnp.float32),
          pltpu.VMEM((2, d), jnp.float32),
          pltpu.SemaphoreType.DMA,
          pltpu.SemaphoreType.DMA,
          pltpu.SemaphoreType.DMA,
      ],
  )
  def sums_kernel(ids_hbm, table_hbm, out_hbm, idx_all, buf0, buf1, acc,
                  sem0, sem1, sem_out):
    wid = lax.axis_index("s") * _NC + lax.axis_index("c")
    row0 = wid * bpw
    base_ids = pl.multiple_of(row0 * s, 8)
    pltpu.sync_copy(ids_hbm.at[pl.ds(base_ids, bpw * s)], idx_all)

    def fire(local_row, bufref, sem):
      ibase = pl.multiple_of(local_row * s, 8)
      for coff, csz in chunks:
        pltpu.async_copy(
            table_hbm.at[idx_all.at[pl.ds(ibase + coff, csz)]],
            bufref.at[pl.ds(coff, csz)], sem)

    def drain_gather(bufref, sem):
      for coff, csz in chunks:
        pltpu.make_async_copy(
            table_hbm.at[idx_all.at[pl.ds(coff, csz)]],
            bufref.at[pl.ds(coff, csz)], sem).wait()

    def accumulate(bufref, slot):
      def acc_body(j, accs):
        j2 = j * 2
        mid = tuple(accs[h] + bufref[j2, pl.ds(h * _LANES, _LANES)]
                    for h in range(nacc))
        return tuple(mid[h] + bufref[j2 + 1, pl.ds(h * _LANES, _LANES)]
                     for h in range(nacc))

      accs = lax.fori_loop(
          0, s // 2, acc_body,
          tuple(jnp.zeros((_LANES,), jnp.float32) for _ in range(nacc)))
      for h in range(nacc):
        acc[slot, pl.ds(h * _LANES, _LANES)] = accs[h]

    fire(0, buf0, sem0)

    def pair_body(g, carry):
      r0 = g * 2
      fire(r0 + 1, buf1, sem1)
      drain_gather(buf0, sem0)

      @pl.when(g > 0)
      def _():
        pltpu.make_async_copy(acc, out_hbm.at[pl.ds(0, 2)], sem_out).wait()

      accumulate(buf0, 0)

      @pl.when(g < half - 1)
      def _():
        fire(r0 + 2, buf0, sem0)

      drain_gather(buf1, sem1)
      accumulate(buf1, 1)
      pltpu.async_copy(acc, out_hbm.at[pl.ds(row0 + r0, 2)], sem_out)
      return carry

    lax.fori_loop(0, half, pair_body, 0)
    pltpu.make_async_copy(acc, out_hbm.at[pl.ds(0, 2)], sem_out).wait()

  return sums_kernel(ids_flat, table)


def _post_tc(sums, input_ids, t0, b, s, h):
  bb = 256
  grid = (b // bb,)
  inv_s = 1.0 / s

  def post_kernel(sums_ref, ids_ref, t0_ref, outr_ref, outi_ref, prob_ref,
                  phase_ref, ampr_ref, ampi_ref):
    ids = ids_ref[...]
    nz = jnp.sum((ids == 0).astype(jnp.float32), axis=1, keepdims=True)
    cnt = s - nz
    tot = sums_ref[...]
    tr = tot[:, :h]
    ti = tot[:, h:]
    t0v = t0_ref[...]
    t0r = t0v[:, :h]
    t0i = t0v[:, h:]
    good = (cnt > 0.0).astype(jnp.float32)
    scale = good / (cnt + 1e-9)
    pr = (tr - nz * t0r) * scale
    pi = (ti - nz * t0i) * scale
    s2 = jnp.sum(pr * pr + pi * pi, axis=1, keepdims=True)
    norm = jnp.sqrt(s2) + 1e-9
    outr = pr / norm
    outi = pi / norm
    outr_ref[...] = outr
    outi_ref[...] = outi
    prob_ref[...] = outr * outr + outi * outi
    phase_ref[...] = jnp.arctan2(outi, outr)
    ampr_ref[...] = tr * inv_s
    ampi_ref[...] = ti * inv_s

  d = 2 * h
  out_spec = pl.BlockSpec((bb, h), lambda i: (i, 0))
  return pl.pallas_call(
      post_kernel,
      grid=grid,
      in_specs=[
          pl.BlockSpec((bb, d), lambda i: (i, 0)),
          pl.BlockSpec((bb, s), lambda i: (i, 0)),
          pl.BlockSpec((1, d), lambda i: (0, 0)),
      ],
      out_specs=[out_spec] * 6,
      out_shape=[jax.ShapeDtypeStruct((b, h), jnp.float32)] * 6,
  )(sums, input_ids, t0)


def kernel(input_ids, word_table):
  b, s = input_ids.shape
  v, d = word_table.shape
  h = d // 2
  ids_flat = input_ids.reshape(-1).astype(jnp.int32)
  sums = _row_sums_sc(ids_flat, word_table, b, s, d)
  outr, outi, prob, phase, ampr, ampi = _post_tc(
      sums, input_ids.astype(jnp.int32), word_table[0:1], b, s, h)
  state = jax.lax.complex(outr, outi)
  amplitudes = jnp.stack([ampr, ampi], axis=-1)
  return (state, amplitudes, prob, phase)
